# baseline (speedup 1.0000x reference)
import functools

import jax
import jax.numpy as jnp
from jax import lax
from jax.experimental import pallas as pl
from jax.experimental.pallas import tpu as pltpu
from jax.experimental.pallas import tpu_sc as plsc

N = 10000
NPAD = 10240
E = 160000
EPAD = 163840
EROWS = EPAD // 128
HID = 512
NCHUNK = 4
CW = 128
BLK = 1000
GRID = N // BLK


def _gumbel_mask(concrete, temp, hard_):
    u = jax.random.uniform(jax.random.key(42), concrete.shape,
                           minval=1e-10, maxval=1.0)
    g = -jnp.log(-jnp.log(u))
    tau = jnp.asarray(temp, concrete.dtype)
    y_soft = jax.nn.softmax((concrete + g) / tau, axis=-1)
    idx = jnp.argmax(y_soft, axis=-1)
    y_hard = jax.nn.one_hot(idx, concrete.shape[-1], dtype=y_soft.dtype)
    y_st = y_hard - lax.stop_gradient(y_soft) + y_soft
    y = jnp.where(jnp.asarray(hard_) != 0, y_st, y_soft)
    return jnp.clip(jnp.sum(y, axis=0), 0.0, 1.0)


def _sc_degree_call(packed, ones_c, zeros_c):
    mesh = plsc.VectorSubcoreMesh(core_axis_name="c", subcore_axis_name="s")

    @functools.partial(
        pl.kernel, mesh=mesh,
        out_type=jax.ShapeDtypeStruct((2 * NPAD, CW), jnp.float32),
        scratch_types=[
            pltpu.VMEM((5120,), jnp.int32),
            pltpu.VMEM((40, 128), jnp.int32),
            pltpu.VMEM((128, CW), jnp.float32),
            pltpu.VMEM_SHARED((NPAD, CW), jnp.float32),
            pltpu.SemaphoreType.DMA,
        ])
    def k(pck_hbm, ones_hbm, zeros_hbm, degp_hbm, pckv, dstv, ov, acc, sem):
        c = lax.axis_index("c")
        s = lax.axis_index("s")
        pltpu.sync_copy(ones_hbm, ov)
        pltpu.sync_copy(pck_hbm.at[pl.ds((c * 16 + s) * 5120, 5120)], pckv)

        def ubody(i, carry):
            v = pckv[pl.ds(i * 16, 16)]
            dstv[i // 8, pl.ds((i % 8) * 16, 16)] = (
                lax.shift_right_logical(v, 16))
            return carry
        lax.fori_loop(0, 320, ubody, 0)

        def zbody(j, carry):
            pltpu.sync_copy(zeros_hbm,
                            acc.at[pl.ds(s * 640 + j * 128, 128)])
            return carry
        lax.fori_loop(0, 5, zbody, 0)
        plsc.subcore_barrier()

        def body(b, carry):
            pltpu.sync_copy(ov, acc.at[dstv.at[b]], add=True)
            return carry
        lax.fori_loop(0, 40, body, 0)
        plsc.subcore_barrier()
        pltpu.sync_copy(acc.at[pl.ds(s * 640, 640)],
                        degp_hbm.at[pl.ds(c * NPAD + s * 640, 640)])

    return k(packed, ones_c, zeros_c)


def _sc_scatter_call(p_flat, packed, zeros_c):
    mesh = plsc.VectorSubcoreMesh(core_axis_name="c", subcore_axis_name="s")

    @functools.partial(
        pl.kernel, mesh=mesh,
        out_type=jax.ShapeDtypeStruct((NCHUNK * NPAD, CW), jnp.float32),
        scratch_types=[
            pltpu.VMEM((10240,), jnp.int32),
            pltpu.VMEM((128,), jnp.int32),
            pltpu.VMEM((128,), jnp.int32),
            pltpu.VMEM((128,), jnp.int32),
            pltpu.VMEM((128, CW), jnp.float32),
            pltpu.VMEM((128, CW), jnp.float32),
            pltpu.SemaphoreType.DMA,
            pltpu.SemaphoreType.DMA,
            pltpu.VMEM_SHARED((NPAD, CW), jnp.float32),
        ])
    def k(pck_hbm, p_hbm, zeros_hbm, seg_hbm,
          pckv, idxa, idxb, dstb, rowsa, rowsb, g0, g1, acc):
        c = lax.axis_index("c")
        s = lax.axis_index("s")
        pltpu.sync_copy(pck_hbm.at[pl.ds(s * 10240, 10240)], pckv)

        def gidx(dst_ref, b, off):
            for kk in range(8):
                v = pckv[pl.ds(b * 128 + kk * 16, 16)]
                dst_ref[pl.ds(kk * 16, 16)] = (
                    jnp.bitwise_and(v, 0xFFFF) + off)

        def didx(b):
            for kk in range(8):
                v = pckv[pl.ds(b * 128 + kk * 16, 16)]
                dstb[pl.ds(kk * 16, 16)] = lax.shift_right_logical(v, 16)

        def chunk_body(lc, carry):
            chunk = 2 * c + lc
            off = chunk * N

            def zbody(j, cy):
                pltpu.sync_copy(zeros_hbm,
                                acc.at[pl.ds(s * 640 + j * 128, 128)])
                return cy
            lax.fori_loop(0, 5, zbody, 0)
            plsc.subcore_barrier()

            gidx(idxa, 0, off)
            pltpu.async_copy(p_hbm.at[idxa], rowsa, g0)

            def pair(t, cy):
                b0i = 2 * t
                gidx(idxb, b0i + 1, off)
                pltpu.async_copy(p_hbm.at[idxb], rowsb, g1)
                pltpu.make_async_copy(p_hbm.at[idxa], rowsa, g0).wait()
                didx(b0i)
                pltpu.sync_copy(rowsa, acc.at[dstb], add=True)

                @pl.when(t < 39)
                def _():
                    gidx(idxa, b0i + 2, off)
                    pltpu.async_copy(p_hbm.at[idxa], rowsa, g0)
                pltpu.make_async_copy(p_hbm.at[idxb], rowsb, g1).wait()
                didx(b0i + 1)
                pltpu.sync_copy(rowsb, acc.at[dstb], add=True)
                return cy
            lax.fori_loop(0, 40, pair, 0)
            plsc.subcore_barrier()
            pltpu.sync_copy(acc.at[pl.ds(s * 640, 640)],
                            seg_hbm.at[pl.ds(chunk * NPAD + s * 640, 640)])
            plsc.subcore_barrier()
            return carry
        lax.fori_loop(0, 2, chunk_body, 0)

    return k(packed, p_flat, zeros_c)


def _cat(ref):
    v = ref[...]
    return jnp.concatenate([v[i] for i in range(NCHUNK)], axis=-1)


def _layernorm(t, g, b):
    mu = jnp.mean(t, axis=-1, keepdims=True)
    var = jnp.mean((t - mu) ** 2, axis=-1, keepdims=True)
    return (t - mu) / jnp.sqrt(var + 1e-5) * g + b


def _dot(a, b):
    return jnp.dot(a, b, preferred_element_type=jnp.float32)


def _k1a_body(x_ref, mask_ref, Win_ref, bin_ref, h0_ref):
    xm = x_ref[...] * mask_ref[...]
    h0_ref[...] = _dot(xm, Win_ref[...]) + bin_ref[...]


def _k1b_body(degp_ref, h0_ref, Wg0_ref, p0_ref, dis_ref):
    dp = degp_ref[...]
    deg = dp[0][:, :1] + dp[1][:, :1] + 1.0
    dis = 1.0 / jnp.sqrt(deg)
    p0 = dis * _dot(h0_ref[...], Wg0_ref[...])
    for ci in range(NCHUNK):
        p0_ref[ci] = p0[:, ci * CW:(ci + 1) * CW]
    dis_ref[...] = jnp.broadcast_to(dis, (BLK, 128))


def _lin_body(h_ref, W_ref, b_ref, o_ref):
    o_ref[...] = _dot(h_ref[...], W_ref[...]) + b_ref[...]


def _k2b_body(seg_ref, p_ref, dis_ref, bg_ref, hl_ref, lg_ref, lb_ref,
              Wg1_ref, Wl1_ref, bl1_ref, h1_ref, p1_ref, hl1_ref):
    dis = dis_ref[...][:, :1]
    t = dis * (_cat(seg_ref) + _cat(p_ref)) + bg_ref[...] + hl_ref[...]
    h1 = jnp.maximum(_layernorm(t, lg_ref[...], lb_ref[...]), 0.0)
    h1_ref[...] = h1
    q = dis * _dot(h1, Wg1_ref[...])
    for ci in range(NCHUNK):
        p1_ref[ci] = q[:, ci * CW:(ci + 1) * CW]
    hl1_ref[...] = _dot(h1, Wl1_ref[...]) + bl1_ref[...]


def _kr_body(x_ref, mask_ref, xyz_ref, Wres_ref, bres_ref, Wxyz_ref,
             bxyz_ref, r_ref):
    xm = x_ref[...] * mask_ref[...]
    r_ref[...] = (_dot(xm, Wres_ref[...]) + bres_ref[...]
                  + _dot(xyz_ref[...], Wxyz_ref[...]) + bxyz_ref[...])


def _k3_body(seg_ref, p_ref, hl1_ref, h1_ref, dis_ref, bg_ref,
             lg_ref, lb_ref, Wp_ref, bp_ref, r_ref, out_ref):
    dis = dis_ref[...][:, :1]
    t = (dis * (_cat(seg_ref) + _cat(p_ref)) + bg_ref[...] + hl1_ref[...])
    h2 = jnp.maximum(_layernorm(t, lg_ref[...], lb_ref[...]), 0.0)
    xf = h1_ref[...] + h2
    out_ref[...] = _dot(xf, Wp_ref[...]) + bp_ref[...] + r_ref[...]


def _row_spec(cols):
    return pl.BlockSpec((BLK, cols), lambda i: (i, 0))


def _full_spec(shape):
    nd = len(shape)
    return pl.BlockSpec(shape, lambda i, _n=nd: (0,) * _n)


def _cm_spec():
    return pl.BlockSpec((NCHUNK, BLK, CW), lambda i: (0, i, 0))


def kernel(x, edge_index, xyz, temp, hard_, concrete, W_in, b_in, Wg0, bg0,
           Wl0, bl0, ln_g0, ln_b0, Wg1, bg1, Wl1, bl1, ln_g1, ln_b1, Wp, bp,
           Wres, bres, Wxyz, bxyz):
    f32 = jnp.float32
    mask = _gumbel_mask(concrete, temp, hard_)[None, :]

    src = edge_index[0]
    dst = edge_index[1]
    pad = EPAD - E
    srcp = jnp.concatenate([src, jnp.zeros((pad,), jnp.int32)])
    dstp = jnp.concatenate([dst, jnp.full((pad,), N, jnp.int32)])
    packed = jnp.bitwise_or(srcp, dstp << 16)

    zeros_c = jnp.zeros((128, CW), f32)
    ones_c = jnp.ones((128, CW), f32)
    xyzp = jnp.pad(xyz, ((0, 0), (0, 126)))
    Wxyzp = jnp.pad(Wxyz, ((0, 126), (0, 0)))

    b_in2 = b_in[None, :]
    bg0_2, bl0_2, lg0_2, lb0_2 = (v[None, :] for v in (bg0, bl0, ln_g0, ln_b0))
    bg1_2, bl1_2, lg1_2, lb1_2 = (v[None, :] for v in (bg1, bl1, ln_g1, ln_b1))
    bp2, bres2, bxyz2 = bp[None, :], bres[None, :], bxyz[None, :]

    degp = _sc_degree_call(packed, ones_c, zeros_c).reshape(2, NPAD, CW)

    h0 = pl.pallas_call(
        _k1a_body, grid=(GRID,),
        in_specs=[_row_spec(256), _full_spec((1, 256)),
                  _full_spec((256, HID)), _full_spec((1, HID))],
        out_specs=_row_spec(HID),
        out_shape=jax.ShapeDtypeStruct((N, HID), f32),
    )(x, mask, W_in, b_in2)

    p0, dis = pl.pallas_call(
        _k1b_body, grid=(GRID,),
        in_specs=[pl.BlockSpec((2, BLK, CW), lambda i: (0, i, 0)),
                  _row_spec(HID), _full_spec((HID, HID))],
        out_specs=[_cm_spec(), _row_spec(128)],
        out_shape=[jax.ShapeDtypeStruct((NCHUNK, N, CW), f32),
                   jax.ShapeDtypeStruct((N, 128), f32)],
    )(degp, h0, Wg0)

    seg0 = _sc_scatter_call(p0.reshape(NCHUNK * N, CW), packed,
                            zeros_c).reshape(NCHUNK, NPAD, CW)

    hl0 = pl.pallas_call(
        _lin_body, grid=(GRID,),
        in_specs=[_row_spec(HID), _full_spec((HID, HID)),
                  _full_spec((1, HID))],
        out_specs=_row_spec(HID),
        out_shape=jax.ShapeDtypeStruct((N, HID), f32),
    )(h0, Wl0, bl0_2)

    h1, p1, hl1 = pl.pallas_call(
        _k2b_body, grid=(GRID,),
        in_specs=[_cm_spec(), _cm_spec(), _row_spec(128),
                  _full_spec((1, HID)), _row_spec(HID),
                  _full_spec((1, HID)), _full_spec((1, HID)),
                  _full_spec((HID, HID)), _full_spec((HID, HID)),
                  _full_spec((1, HID))],
        out_specs=[_row_spec(HID), _cm_spec(), _row_spec(HID)],
        out_shape=[jax.ShapeDtypeStruct((N, HID), f32),
                   jax.ShapeDtypeStruct((NCHUNK, N, CW), f32),
                   jax.ShapeDtypeStruct((N, HID), f32)],
    )(seg0, p0, dis, bg0_2, hl0, lg0_2, lb0_2, Wg1, Wl1, bl1_2)

    seg1 = _sc_scatter_call(p1.reshape(NCHUNK * N, CW), packed,
                            zeros_c).reshape(NCHUNK, NPAD, CW)

    r = pl.pallas_call(
        _kr_body, grid=(GRID,),
        in_specs=[_row_spec(256), _full_spec((1, 256)), _row_spec(128),
                  _full_spec((256, 256)), _full_spec((1, 256)),
                  _full_spec((128, 256)), _full_spec((1, 256))],
        out_specs=_row_spec(256),
        out_shape=jax.ShapeDtypeStruct((N, 256), f32),
    )(x, mask, xyzp, Wres, bres2, Wxyzp, bxyz2)

    out = pl.pallas_call(
        _k3_body, grid=(GRID,),
        in_specs=[_cm_spec(), _cm_spec(), _row_spec(HID), _row_spec(HID),
                  _row_spec(128), _full_spec((1, HID)),
                  _full_spec((1, HID)), _full_spec((1, HID)),
                  _full_spec((HID, 256)), _full_spec((1, 256)),
                  _row_spec(256)],
        out_specs=_row_spec(256),
        out_shape=jax.ShapeDtypeStruct((N, 256), f32),
    )(seg1, p1, hl1, h1, dis, bg1_2, lg1_2, lb1_2, Wp, bp2, r)

    return out


# speedup vs baseline: 1.0406x; 1.0406x over previous
import functools

import jax
import jax.numpy as jnp
from jax import lax
from jax.experimental import pallas as pl
from jax.experimental.pallas import tpu as pltpu
from jax.experimental.pallas import tpu_sc as plsc

N = 10000
NPAD = 10240
E = 160000
EPAD = 163840
EROWS = EPAD // 128
HID = 512
NCHUNK = 4
CW = 128
BLK = 1000
GRID = N // BLK


def _gumbel_mask(concrete, temp, hard_):
    u = jax.random.uniform(jax.random.key(42), concrete.shape,
                           minval=1e-10, maxval=1.0)
    g = -jnp.log(-jnp.log(u))
    tau = jnp.asarray(temp, concrete.dtype)
    y_soft = jax.nn.softmax((concrete + g) / tau, axis=-1)
    idx = jnp.argmax(y_soft, axis=-1)
    y_hard = jax.nn.one_hot(idx, concrete.shape[-1], dtype=y_soft.dtype)
    y_st = y_hard - lax.stop_gradient(y_soft) + y_soft
    y = jnp.where(jnp.asarray(hard_) != 0, y_st, y_soft)
    return jnp.clip(jnp.sum(y, axis=0), 0.0, 1.0)


def _sc_degree_call(packed, ones_c, zeros_c):
    mesh = plsc.VectorSubcoreMesh(core_axis_name="c", subcore_axis_name="s")

    @functools.partial(
        pl.kernel, mesh=mesh,
        out_type=jax.ShapeDtypeStruct((2 * NPAD, CW), jnp.float32),
        scratch_types=[
            pltpu.VMEM((5120,), jnp.int32),
            pltpu.VMEM((40, 128), jnp.int32),
            pltpu.VMEM((128, CW), jnp.float32),
            pltpu.VMEM_SHARED((NPAD, CW), jnp.float32),
            pltpu.SemaphoreType.DMA,
        ])
    def k(pck_hbm, ones_hbm, zeros_hbm, degp_hbm, pckv, dstv, ov, acc, sem):
        c = lax.axis_index("c")
        s = lax.axis_index("s")
        pltpu.sync_copy(ones_hbm, ov)
        pltpu.sync_copy(pck_hbm.at[pl.ds((c * 16 + s) * 5120, 5120)], pckv)

        def ubody(i, carry):
            v = pckv[pl.ds(i * 16, 16)]
            dstv[i // 8, pl.ds((i % 8) * 16, 16)] = (
                lax.shift_right_logical(v, 16))
            return carry
        lax.fori_loop(0, 320, ubody, 0)

        def zbody(j, carry):
            pltpu.sync_copy(zeros_hbm,
                            acc.at[pl.ds(s * 640 + j * 128, 128)])
            return carry
        lax.fori_loop(0, 5, zbody, 0)
        plsc.subcore_barrier()

        def body(b, carry):
            pltpu.sync_copy(ov, acc.at[dstv.at[b]], add=True)
            return carry
        lax.fori_loop(0, 40, body, 0)
        plsc.subcore_barrier()
        pltpu.sync_copy(acc.at[pl.ds(s * 640, 640)],
                        degp_hbm.at[pl.ds(c * NPAD + s * 640, 640)])

    return k(packed, ones_c, zeros_c)


def _sc_scatter_call(p_flat, packed, zeros_c):
    mesh = plsc.VectorSubcoreMesh(core_axis_name="c", subcore_axis_name="s")

    @functools.partial(
        pl.kernel, mesh=mesh,
        out_type=jax.ShapeDtypeStruct((NCHUNK * NPAD, CW), jnp.float32),
        scratch_types=[
            pltpu.VMEM((10240,), jnp.int32),
            pltpu.VMEM((128,), jnp.int32),
            pltpu.VMEM((128,), jnp.int32),
            pltpu.VMEM((128,), jnp.int32),
            pltpu.VMEM((128, CW), jnp.float32),
            pltpu.VMEM((128, CW), jnp.float32),
            pltpu.SemaphoreType.DMA,
            pltpu.SemaphoreType.DMA,
            pltpu.VMEM_SHARED((NPAD, CW), jnp.float32),
        ])
    def k(pck_hbm, p_hbm, zeros_hbm, seg_hbm,
          pckv, idxa, idxb, dstb, rowsa, rowsb, g0, g1, acc):
        c = lax.axis_index("c")
        s = lax.axis_index("s")
        pltpu.sync_copy(pck_hbm.at[pl.ds(s * 10240, 10240)], pckv)

        def gidx(dst_ref, b, off):
            for kk in range(8):
                v = pckv[pl.ds(b * 128 + kk * 16, 16)]
                dst_ref[pl.ds(kk * 16, 16)] = (
                    jnp.bitwise_and(v, 0xFFFF) + off)

        def didx(b):
            for kk in range(8):
                v = pckv[pl.ds(b * 128 + kk * 16, 16)]
                dstb[pl.ds(kk * 16, 16)] = lax.shift_right_logical(v, 16)

        def chunk_body(lc, carry):
            chunk = 2 * c + lc
            off = chunk * N

            def zbody(j, cy):
                pltpu.sync_copy(zeros_hbm,
                                acc.at[pl.ds(s * 640 + j * 128, 128)])
                return cy
            lax.fori_loop(0, 5, zbody, 0)
            plsc.subcore_barrier()

            gidx(idxa, 0, off)
            pltpu.async_copy(p_hbm.at[idxa], rowsa, g0)

            def pair(t, cy):
                b0i = 2 * t
                gidx(idxb, b0i + 1, off)
                pltpu.async_copy(p_hbm.at[idxb], rowsb, g1)
                pltpu.make_async_copy(p_hbm.at[idxa], rowsa, g0).wait()
                didx(b0i)
                pltpu.sync_copy(rowsa, acc.at[dstb], add=True)

                @pl.when(t < 39)
                def _():
                    gidx(idxa, b0i + 2, off)
                    pltpu.async_copy(p_hbm.at[idxa], rowsa, g0)
                pltpu.make_async_copy(p_hbm.at[idxb], rowsb, g1).wait()
                didx(b0i + 1)
                pltpu.sync_copy(rowsb, acc.at[dstb], add=True)
                return cy
            lax.fori_loop(0, 40, pair, 0)
            plsc.subcore_barrier()
            pltpu.sync_copy(acc.at[pl.ds(s * 640, 640)],
                            seg_hbm.at[pl.ds(chunk * NPAD + s * 640, 640)])
            plsc.subcore_barrier()
            return carry
        lax.fori_loop(0, 2, chunk_body, 0)

    return k(packed, p_flat, zeros_c)


def _cat(ref):
    v = ref[...]
    return jnp.concatenate([v[i] for i in range(NCHUNK)], axis=-1)


def _layernorm(t, g, b):
    mu = jnp.mean(t, axis=-1, keepdims=True)
    var = jnp.mean((t - mu) ** 2, axis=-1, keepdims=True)
    return (t - mu) / jnp.sqrt(var + 1e-5) * g + b


def _dot(a, b):
    return jnp.dot(a, b, preferred_element_type=jnp.float32)


def _k1_body(x_ref, mask_ref, degp_ref, xyz_ref, Win_ref, bin_ref, Wg0_ref,
             Wres_ref, bres_ref, Wxyz_ref, bxyz_ref,
             h0_ref, p0_ref, dis_ref, r_ref):
    xm = x_ref[...] * mask_ref[...]
    h0 = _dot(xm, Win_ref[...]) + bin_ref[...]
    dp = degp_ref[...]
    deg = dp[0][:, :1] + dp[1][:, :1] + 1.0
    dis = 1.0 / jnp.sqrt(deg)
    p0 = dis * _dot(h0, Wg0_ref[...])
    r = (_dot(xm, Wres_ref[...]) + bres_ref[...]
         + _dot(xyz_ref[...], Wxyz_ref[...]) + bxyz_ref[...])
    h0_ref[...] = h0
    for ci in range(NCHUNK):
        p0_ref[ci] = p0[:, ci * CW:(ci + 1) * CW]
    dis_ref[...] = jnp.broadcast_to(dis, (BLK, 128))
    r_ref[...] = r


def _k2_body(seg_ref, p_ref, h_ref, dis_ref, bg_ref, Wl_ref, bl_ref,
             lg_ref, lb_ref, Wg1_ref, h1_ref, p1_ref):
    dis = dis_ref[...][:, :1]
    conv = dis * (_cat(seg_ref) + _cat(p_ref)) + bg_ref[...]
    t = conv + _dot(h_ref[...], Wl_ref[...]) + bl_ref[...]
    h1 = jnp.maximum(_layernorm(t, lg_ref[...], lb_ref[...]), 0.0)
    h1_ref[...] = h1
    q = dis * _dot(h1, Wg1_ref[...])
    for ci in range(NCHUNK):
        p1_ref[ci] = q[:, ci * CW:(ci + 1) * CW]


def _k3_body(seg_ref, p_ref, h_ref, dis_ref, bg_ref, Wl_ref, bl_ref,
             lg_ref, lb_ref, Wp_ref, bp_ref, r_ref, out_ref):
    dis = dis_ref[...][:, :1]
    conv = dis * (_cat(seg_ref) + _cat(p_ref)) + bg_ref[...]
    t = conv + _dot(h_ref[...], Wl_ref[...]) + bl_ref[...]
    h2 = jnp.maximum(_layernorm(t, lg_ref[...], lb_ref[...]), 0.0)
    xf = h_ref[...] + h2
    out_ref[...] = _dot(xf, Wp_ref[...]) + bp_ref[...] + r_ref[...]


def _row_spec(cols):
    return pl.BlockSpec((BLK, cols), lambda i: (i, 0))


def _full_spec(shape):
    nd = len(shape)
    return pl.BlockSpec(shape, lambda i, _n=nd: (0,) * _n)


def _cm_spec():
    return pl.BlockSpec((NCHUNK, BLK, CW), lambda i: (0, i, 0))


def kernel(x, edge_index, xyz, temp, hard_, concrete, W_in, b_in, Wg0, bg0,
           Wl0, bl0, ln_g0, ln_b0, Wg1, bg1, Wl1, bl1, ln_g1, ln_b1, Wp, bp,
           Wres, bres, Wxyz, bxyz):
    f32 = jnp.float32
    mask = _gumbel_mask(concrete, temp, hard_)[None, :]

    src = edge_index[0]
    dst = edge_index[1]
    pad = EPAD - E
    srcp = jnp.concatenate([src, jnp.zeros((pad,), jnp.int32)])
    dstp = jnp.concatenate([dst, jnp.full((pad,), N, jnp.int32)])
    packed = jnp.bitwise_or(srcp, dstp << 16)

    zeros_c = jnp.zeros((128, CW), f32)
    ones_c = jnp.ones((128, CW), f32)
    xyzp = jnp.pad(xyz, ((0, 0), (0, 126)))
    Wxyzp = jnp.pad(Wxyz, ((0, 126), (0, 0)))

    b_in2 = b_in[None, :]
    bg0_2, bl0_2, lg0_2, lb0_2 = (v[None, :] for v in (bg0, bl0, ln_g0, ln_b0))
    bg1_2, bl1_2, lg1_2, lb1_2 = (v[None, :] for v in (bg1, bl1, ln_g1, ln_b1))
    bp2, bres2, bxyz2 = bp[None, :], bres[None, :], bxyz[None, :]

    degp = _sc_degree_call(packed, ones_c, zeros_c).reshape(2, NPAD, CW)

    h0, p0, dis, r = pl.pallas_call(
        _k1_body,
        grid=(GRID,),
        in_specs=[
            _row_spec(256), _full_spec((1, 256)),
            pl.BlockSpec((2, BLK, CW), lambda i: (0, i, 0)),
            _row_spec(128),
            _full_spec((256, HID)), _full_spec((1, HID)),
            _full_spec((HID, HID)),
            _full_spec((256, 256)), _full_spec((1, 256)),
            _full_spec((128, 256)), _full_spec((1, 256)),
        ],
        out_specs=[_row_spec(HID), _cm_spec(), _row_spec(128), _row_spec(256)],
        out_shape=[
            jax.ShapeDtypeStruct((N, HID), f32),
            jax.ShapeDtypeStruct((NCHUNK, N, CW), f32),
            jax.ShapeDtypeStruct((N, 128), f32),
            jax.ShapeDtypeStruct((N, 256), f32),
        ],
    )(x, mask, degp, xyzp, W_in, b_in2, Wg0, Wres, bres2, Wxyzp, bxyz2)

    seg0 = _sc_scatter_call(p0.reshape(NCHUNK * N, CW), packed,
                            zeros_c).reshape(NCHUNK, NPAD, CW)

    h1, p1 = pl.pallas_call(
        _k2_body,
        grid=(GRID,),
        in_specs=[
            _cm_spec(),
            _cm_spec(), _row_spec(HID), _row_spec(128),
            _full_spec((1, HID)), _full_spec((HID, HID)), _full_spec((1, HID)),
            _full_spec((1, HID)), _full_spec((1, HID)),
            _full_spec((HID, HID)),
        ],
        out_specs=[_row_spec(HID), _cm_spec()],
        out_shape=[
            jax.ShapeDtypeStruct((N, HID), f32),
            jax.ShapeDtypeStruct((NCHUNK, N, CW), f32),
        ],
    )(seg0, p0, h0, dis, bg0_2, Wl0, bl0_2, lg0_2, lb0_2, Wg1)

    seg1 = _sc_scatter_call(p1.reshape(NCHUNK * N, CW), packed,
                            zeros_c).reshape(NCHUNK, NPAD, CW)

    out = pl.pallas_call(
        _k3_body,
        grid=(GRID,),
        in_specs=[
            _cm_spec(),
            _cm_spec(), _row_spec(HID), _row_spec(128),
            _full_spec((1, HID)), _full_spec((HID, HID)), _full_spec((1, HID)),
            _full_spec((1, HID)), _full_spec((1, HID)),
            _full_spec((HID, 256)), _full_spec((1, 256)),
            _row_spec(256),
        ],
        out_specs=_row_spec(256),
        out_shape=jax.ShapeDtypeStruct((N, 256), f32),
    )(seg1, p1, h1, dis, bg1_2, Wl1, bl1_2, lg1_2, lb1_2, Wp, bp2, r)

    return out


# speedup vs baseline: 1.0583x; 1.0170x over previous
import functools

import jax
import jax.numpy as jnp
from jax import lax
from jax.experimental import pallas as pl
from jax.experimental.pallas import tpu as pltpu
from jax.experimental.pallas import tpu_sc as plsc

N = 10000
NPAD = 10240
E = 160000
EPAD = 163840
EROWS = EPAD // 128
HID = 512
NCHUNK = 4
CW = 128
BLK = 1000
GRID = N // BLK


def _gumbel_mask(concrete, temp, hard_):
    u = jax.random.uniform(jax.random.key(42), concrete.shape,
                           minval=1e-10, maxval=1.0)
    g = -jnp.log(-jnp.log(u))
    tau = jnp.asarray(temp, concrete.dtype)
    y_soft = jax.nn.softmax((concrete + g) / tau, axis=-1)
    idx = jnp.argmax(y_soft, axis=-1)
    y_hard = jax.nn.one_hot(idx, concrete.shape[-1], dtype=y_soft.dtype)
    y_st = y_hard - lax.stop_gradient(y_soft) + y_soft
    y = jnp.where(jnp.asarray(hard_) != 0, y_st, y_soft)
    return jnp.clip(jnp.sum(y, axis=0), 0.0, 1.0)


def _sc_degree_call(packed, ones_c, zeros_c):
    mesh = plsc.VectorSubcoreMesh(core_axis_name="c", subcore_axis_name="s")

    @functools.partial(
        pl.kernel, mesh=mesh,
        out_type=jax.ShapeDtypeStruct((2 * NPAD, CW), jnp.float32),
        scratch_types=[
            pltpu.VMEM((5120,), jnp.int32),
            pltpu.VMEM((40, 128), jnp.int32),
            pltpu.VMEM((128, CW), jnp.float32),
            pltpu.VMEM_SHARED((NPAD, CW), jnp.float32),
            pltpu.SemaphoreType.DMA,
        ])
    def k(pck_hbm, ones_hbm, zeros_hbm, degp_hbm, pckv, dstv, ov, acc, sem):
        c = lax.axis_index("c")
        s = lax.axis_index("s")
        pltpu.sync_copy(ones_hbm, ov)
        pltpu.sync_copy(pck_hbm.at[pl.ds((c * 16 + s) * 5120, 5120)], pckv)

        def ubody(i, carry):
            v = pckv[pl.ds(i * 16, 16)]
            dstv[i // 8, pl.ds((i % 8) * 16, 16)] = (
                lax.shift_right_logical(v, 16))
            return carry
        lax.fori_loop(0, 320, ubody, 0)

        def zbody(j, carry):
            pltpu.sync_copy(zeros_hbm,
                            acc.at[pl.ds(s * 640 + j * 128, 128)])
            return carry
        lax.fori_loop(0, 5, zbody, 0)
        plsc.subcore_barrier()

        def body(b, carry):
            pltpu.sync_copy(ov, acc.at[dstv.at[b]], add=True)
            return carry
        lax.fori_loop(0, 40, body, 0)
        plsc.subcore_barrier()
        pltpu.sync_copy(acc.at[pl.ds(s * 640, 640)],
                        degp_hbm.at[pl.ds(c * NPAD + s * 640, 640)])

    return k(packed, ones_c, zeros_c)


def _sc_scatter_call(p_flat, packed, zeros_c):
    mesh = plsc.VectorSubcoreMesh(core_axis_name="c", subcore_axis_name="s")

    @functools.partial(
        pl.kernel, mesh=mesh,
        out_type=jax.ShapeDtypeStruct((NCHUNK * NPAD, CW), jnp.float32),
        scratch_types=[
            pltpu.VMEM((10240,), jnp.int32),
            pltpu.VMEM((80,), jnp.int32),
            pltpu.VMEM((80,), jnp.int32),
            pltpu.VMEM((80,), jnp.int32),
            pltpu.VMEM((80,), jnp.int32),
            pltpu.VMEM((80, CW), jnp.float32),
            pltpu.VMEM((80, CW), jnp.float32),
            pltpu.VMEM((80, CW), jnp.float32),
            pltpu.SemaphoreType.DMA,
            pltpu.SemaphoreType.DMA,
            pltpu.SemaphoreType.DMA,
            pltpu.VMEM_SHARED((NPAD, CW), jnp.float32),
        ])
    def k(pck_hbm, p_hbm, zeros_hbm, seg_hbm,
          pckv, idx0, idx1, idx2, dstb, rows0, rows1, rows2,
          g0, g1, g2, acc):
        c = lax.axis_index("c")
        s = lax.axis_index("s")
        pltpu.sync_copy(pck_hbm.at[pl.ds(s * 10240, 10240)], pckv)
        idxs = (idx0, idx1, idx2)
        rows = (rows0, rows1, rows2)
        gs = (g0, g1, g2)

        def gidx(dst_ref, b, off):
            for kk in range(5):
                v = pckv[pl.ds(b * 80 + kk * 16, 16)]
                dst_ref[pl.ds(kk * 16, 16)] = (
                    jnp.bitwise_and(v, 0xFFFF) + off)

        def didx(b):
            for kk in range(5):
                v = pckv[pl.ds(b * 80 + kk * 16, 16)]
                dstb[pl.ds(kk * 16, 16)] = lax.shift_right_logical(v, 16)

        def chunk_body(lc, carry):
            chunk = 2 * c + lc
            off = chunk * N

            def zbody(j, cy):
                pltpu.sync_copy(zeros_hbm,
                                acc.at[pl.ds(s * 640 + j * 128, 128)])
                return cy
            lax.fori_loop(0, 5, zbody, 0)
            plsc.subcore_barrier()

            gidx(idx0, 0, off)
            pltpu.async_copy(p_hbm.at[idx0], rows0, g0)
            gidx(idx1, 1, off)
            pltpu.async_copy(p_hbm.at[idx1], rows1, g1)

            def tri(t, cy):
                b0i = 3 * t
                for j in range(3):
                    b = b0i + j
                    nb = b + 2
                    cur = j % 3
                    nxt = (j + 2) % 3

                    @pl.when(nb < 128)
                    def _():
                        gidx(idxs[nxt], nb, off)
                        pltpu.async_copy(p_hbm.at[idxs[nxt]], rows[nxt],
                                         gs[nxt])

                    @pl.when(b < 128)
                    def _():
                        pltpu.make_async_copy(p_hbm.at[idxs[cur]], rows[cur],
                                              gs[cur]).wait()
                        didx(b)
                        pltpu.sync_copy(rows[cur], acc.at[dstb], add=True)
                return cy
            lax.fori_loop(0, 43, tri, 0)
            plsc.subcore_barrier()
            pltpu.sync_copy(acc.at[pl.ds(s * 640, 640)],
                            seg_hbm.at[pl.ds(chunk * NPAD + s * 640, 640)])
            plsc.subcore_barrier()
            return carry
        lax.fori_loop(0, 2, chunk_body, 0)

    return k(packed, p_flat, zeros_c)


def _cat(ref):
    v = ref[...]
    return jnp.concatenate([v[i] for i in range(NCHUNK)], axis=-1)


def _layernorm(t, g, b):
    mu = jnp.mean(t, axis=-1, keepdims=True)
    var = jnp.mean((t - mu) ** 2, axis=-1, keepdims=True)
    return (t - mu) / jnp.sqrt(var + 1e-5) * g + b


def _dot(a, b):
    return jnp.dot(a, b, preferred_element_type=jnp.float32)


def _k1_body(x_ref, mask_ref, degp_ref, xyz_ref, Win_ref, bin_ref, Wg0_ref,
             Wres_ref, bres_ref, Wxyz_ref, bxyz_ref,
             h0_ref, p0_ref, dis_ref, r_ref):
    xm = x_ref[...] * mask_ref[...]
    h0 = _dot(xm, Win_ref[...]) + bin_ref[...]
    dp = degp_ref[...]
    deg = dp[0][:, :1] + dp[1][:, :1] + 1.0
    dis = 1.0 / jnp.sqrt(deg)
    p0 = dis * _dot(h0, Wg0_ref[...])
    r = (_dot(xm, Wres_ref[...]) + bres_ref[...]
         + _dot(xyz_ref[...], Wxyz_ref[...]) + bxyz_ref[...])
    h0_ref[...] = h0
    for ci in range(NCHUNK):
        p0_ref[ci] = p0[:, ci * CW:(ci + 1) * CW]
    dis_ref[...] = jnp.broadcast_to(dis, (BLK, 128))
    r_ref[...] = r


def _k2_body(seg_ref, p_ref, h_ref, dis_ref, bg_ref, Wl_ref, bl_ref,
             lg_ref, lb_ref, Wg1_ref, h1_ref, p1_ref):
    dis = dis_ref[...][:, :1]
    conv = dis * (_cat(seg_ref) + _cat(p_ref)) + bg_ref[...]
    t = conv + _dot(h_ref[...], Wl_ref[...]) + bl_ref[...]
    h1 = jnp.maximum(_layernorm(t, lg_ref[...], lb_ref[...]), 0.0)
    h1_ref[...] = h1
    q = dis * _dot(h1, Wg1_ref[...])
    for ci in range(NCHUNK):
        p1_ref[ci] = q[:, ci * CW:(ci + 1) * CW]


def _k3_body(seg_ref, p_ref, h_ref, dis_ref, bg_ref, Wl_ref, bl_ref,
             lg_ref, lb_ref, Wp_ref, bp_ref, r_ref, out_ref):
    dis = dis_ref[...][:, :1]
    conv = dis * (_cat(seg_ref) + _cat(p_ref)) + bg_ref[...]
    t = conv + _dot(h_ref[...], Wl_ref[...]) + bl_ref[...]
    h2 = jnp.maximum(_layernorm(t, lg_ref[...], lb_ref[...]), 0.0)
    xf = h_ref[...] + h2
    out_ref[...] = _dot(xf, Wp_ref[...]) + bp_ref[...] + r_ref[...]


def _row_spec(cols):
    return pl.BlockSpec((BLK, cols), lambda i: (i, 0))


def _full_spec(shape):
    nd = len(shape)
    return pl.BlockSpec(shape, lambda i, _n=nd: (0,) * _n)


def _cm_spec():
    return pl.BlockSpec((NCHUNK, BLK, CW), lambda i: (0, i, 0))


def kernel(x, edge_index, xyz, temp, hard_, concrete, W_in, b_in, Wg0, bg0,
           Wl0, bl0, ln_g0, ln_b0, Wg1, bg1, Wl1, bl1, ln_g1, ln_b1, Wp, bp,
           Wres, bres, Wxyz, bxyz):
    f32 = jnp.float32
    mask = _gumbel_mask(concrete, temp, hard_)[None, :]

    src = edge_index[0]
    dst = edge_index[1]
    pad = EPAD - E
    srcp = jnp.concatenate([src, jnp.zeros((pad,), jnp.int32)])
    dstp = jnp.concatenate([dst, jnp.full((pad,), N, jnp.int32)])
    packed = jnp.bitwise_or(srcp, dstp << 16)

    zeros_c = jnp.zeros((128, CW), f32)
    ones_c = jnp.ones((128, CW), f32)
    xyzp = jnp.pad(xyz, ((0, 0), (0, 126)))
    Wxyzp = jnp.pad(Wxyz, ((0, 126), (0, 0)))

    b_in2 = b_in[None, :]
    bg0_2, bl0_2, lg0_2, lb0_2 = (v[None, :] for v in (bg0, bl0, ln_g0, ln_b0))
    bg1_2, bl1_2, lg1_2, lb1_2 = (v[None, :] for v in (bg1, bl1, ln_g1, ln_b1))
    bp2, bres2, bxyz2 = bp[None, :], bres[None, :], bxyz[None, :]

    degp = _sc_degree_call(packed, ones_c, zeros_c).reshape(2, NPAD, CW)

    h0, p0, dis, r = pl.pallas_call(
        _k1_body,
        grid=(GRID,),
        in_specs=[
            _row_spec(256), _full_spec((1, 256)),
            pl.BlockSpec((2, BLK, CW), lambda i: (0, i, 0)),
            _row_spec(128),
            _full_spec((256, HID)), _full_spec((1, HID)),
            _full_spec((HID, HID)),
            _full_spec((256, 256)), _full_spec((1, 256)),
            _full_spec((128, 256)), _full_spec((1, 256)),
        ],
        out_specs=[_row_spec(HID), _cm_spec(), _row_spec(128), _row_spec(256)],
        out_shape=[
            jax.ShapeDtypeStruct((N, HID), f32),
            jax.ShapeDtypeStruct((NCHUNK, N, CW), f32),
            jax.ShapeDtypeStruct((N, 128), f32),
            jax.ShapeDtypeStruct((N, 256), f32),
        ],
    )(x, mask, degp, xyzp, W_in, b_in2, Wg0, Wres, bres2, Wxyzp, bxyz2)

    seg0 = _sc_scatter_call(p0.reshape(NCHUNK * N, CW), packed,
                            zeros_c).reshape(NCHUNK, NPAD, CW)

    h1, p1 = pl.pallas_call(
        _k2_body,
        grid=(GRID,),
        in_specs=[
            _cm_spec(),
            _cm_spec(), _row_spec(HID), _row_spec(128),
            _full_spec((1, HID)), _full_spec((HID, HID)), _full_spec((1, HID)),
            _full_spec((1, HID)), _full_spec((1, HID)),
            _full_spec((HID, HID)),
        ],
        out_specs=[_row_spec(HID), _cm_spec()],
        out_shape=[
            jax.ShapeDtypeStruct((N, HID), f32),
            jax.ShapeDtypeStruct((NCHUNK, N, CW), f32),
        ],
    )(seg0, p0, h0, dis, bg0_2, Wl0, bl0_2, lg0_2, lb0_2, Wg1)

    seg1 = _sc_scatter_call(p1.reshape(NCHUNK * N, CW), packed,
                            zeros_c).reshape(NCHUNK, NPAD, CW)

    out = pl.pallas_call(
        _k3_body,
        grid=(GRID,),
        in_specs=[
            _cm_spec(),
            _cm_spec(), _row_spec(HID), _row_spec(128),
            _full_spec((1, HID)), _full_spec((HID, HID)), _full_spec((1, HID)),
            _full_spec((1, HID)), _full_spec((1, HID)),
            _full_spec((HID, 256)), _full_spec((1, 256)),
            _row_spec(256),
        ],
        out_specs=_row_spec(256),
        out_shape=jax.ShapeDtypeStruct((N, 256), f32),
    )(seg1, p1, h1, dis, bg1_2, Wl1, bl1_2, lg1_2, lb1_2, Wp, bp2, r)

    return out
